# Initial kernel scaffold; baseline (speedup 1.0000x reference)
import jax
import jax.numpy as jnp
from jax.experimental import pallas as pl


def kernel(n_feats, edge_index, e_weights, W, b):
    raise NotImplementedError("write your pallas kernel here")


# speedup vs baseline: 1.5874x; 1.5874x over previous
import dataclasses
import functools

import jax
import jax.numpy as jnp
from jax import lax
from jax.experimental import pallas as pl
from jax.experimental.pallas import tpu as pltpu
from jax.experimental.pallas import tpu_sc as plsc

N = 10000
E = 160000
D = 256
HALF = 128
NPAD = 10240
C = 80
NSUB = 16
EPS = E // NSUB
NCHUNK = EPS // C
NHALF = NPAD // 2
NPASS = 2
PASSN = NHALF
ACCR = PASSN + 16
APS = ACCR // NSUB
CPS = PASSN // NSUB
DNODE = NHALF // NPASS
DROWS = DNODE * 16 // 128
DROWSP = 384
DPS = DROWS // NSUB


def _sc_segment_sum(nf2, src, dst, w):
    mesh = plsc.VectorSubcoreMesh(core_axis_name="c", subcore_axis_name="s")
    cp = pltpu.CompilerParams()
    if "needs_layout_passes" in pltpu.CompilerParams.__dataclass_fields__:
        cp = dataclasses.replace(cp, needs_layout_passes=False)

    @functools.partial(
        pl.kernel,
        compiler_params=cp,
        out_type=(
            jax.ShapeDtypeStruct((2, NPAD, HALF), jnp.float32),
            jax.ShapeDtypeStruct((2, NPASS * DROWSP, 128), jnp.float32),
        ),
        mesh=mesh,
        scratch_types=[
            pltpu.VMEM((C,), jnp.int32),
            pltpu.VMEM((C,), jnp.int32),
            pltpu.VMEM((C,), jnp.int32),
            pltpu.VMEM((C,), jnp.int32),
            pltpu.VMEM((C,), jnp.float32),
            pltpu.VMEM((C, HALF), jnp.float32),
            pltpu.VMEM((C, HALF), jnp.float32),
            pltpu.VMEM((DROWSP, 128), jnp.float32),
            pltpu.VMEM((DROWSP // 128, 128), jnp.int32),
            pltpu.VMEM_SHARED((ACCR, HALF), jnp.float32),
            pltpu.VMEM_SHARED((DROWSP, 128), jnp.float32),
            pltpu.SemaphoreType.DMA,
        ],
    )
    def k(nf2_hbm, src_hbm, dst_hbm, w_hbm, feat_hbm, deg_hbm,
          src_v, adj_v, dst_v, rel_v, w_v, rows_v, msg_v, degh_v, id_v,
          acc_sh, deg_sh, sem):
        cid = lax.axis_index("c")
        sid = lax.axis_index("s")
        zero16 = jnp.zeros((16,), jnp.float32)
        one16 = jnp.ones((16,), jnp.float32)
        lane16 = jnp.arange(16, dtype=jnp.int32)
        ebase = sid * EPS
        off = cid * N

        for j in range(DROWSP // 128):
            for t in range(128 // 16):
                id_v[j, pl.ds(t * 16, 16)] = lane16 + (j * 128 + t * 16)

        for p in range(NPASS):
            acc_lo = p * PASSN
            deg_lo = cid * NHALF + p * DNODE

            @pl.loop(0, DROWSP)
            def _(r):
                for j in range(128 // 16):
                    degh_v[r, pl.ds(j * 16, 16)] = zero16

            @pl.loop(0, C)
            def _(r):
                for j in range(HALF // 16):
                    msg_v[r, pl.ds(j * 16, 16)] = zero16

            abase = sid * APS
            for t in range(APS // C):
                pltpu.sync_copy(msg_v, acc_sh.at[pl.ds(abase + t * C, C)])
            pltpu.sync_copy(msg_v.at[pl.ds(0, APS % C)],
                            acc_sh.at[pl.ds(abase + (APS // C) * C, APS % C)])
            dzps = DROWSP // NSUB
            pltpu.sync_copy(msg_v.at[pl.ds(0, dzps)],
                            deg_sh.at[pl.ds(sid * dzps, dzps)])

            plsc.subcore_barrier()

            @pl.loop(0, NCHUNK)
            def _(kk):
                b = ebase + kk * C
                pltpu.sync_copy(src_hbm.at[pl.ds(b, C)], src_v)
                pltpu.sync_copy(dst_hbm.at[pl.ds(b, C)], dst_v)
                pltpu.sync_copy(w_hbm.at[pl.ds(b, C)], w_v)

                @pl.loop(0, C, step=16)
                def _(i):
                    adj_v[pl.ds(i, 16)] = src_v[pl.ds(i, 16)] + off
                    d16 = dst_v[pl.ds(i, 16)]
                    rel = d16 - acc_lo
                    mask = (rel >= 0) & (rel < PASSN)
                    rel_v[pl.ds(i, 16)] = jnp.where(mask, rel, PASSN)

                pltpu.async_copy(nf2_hbm.at[adj_v], rows_v, sem).wait()

                @pl.loop(0, C, step=16)
                def _(i):
                    w16 = w_v[pl.ds(i, 16)]
                    for r in range(16):
                        wv = jnp.full((16,), w16[r], jnp.float32)
                        for j in range(HALF // 16):
                            msg_v[i + r, pl.ds(j * 16, 16)] = (
                                rows_v[i + r, pl.ds(j * 16, 16)] * wv)

                pltpu.sync_copy(msg_v, acc_sh.at[rel_v], add=True)

                @pl.loop(0, C, step=16)
                def _(i):
                    d16 = dst_v[pl.ds(i, 16)]
                    rel = d16 - deg_lo
                    mask = (rel >= 0) & (rel < DNODE)
                    relc = jnp.where(mask, rel, 0)
                    flat = relc * 16 + lane16
                    plsc.addupdate_scatter(
                        degh_v, [flat >> 7, flat & 127], one16, mask=mask)

            for j in range(DROWSP // 128):
                pltpu.sync_copy(degh_v.at[pl.ds(j * 128, 128)],
                                deg_sh.at[id_v.at[j]], add=True)

            plsc.subcore_barrier()

            for t in range(CPS // C):
                pltpu.sync_copy(
                    acc_sh.at[pl.ds(sid * CPS + t * C, C)],
                    feat_hbm.at[cid].at[
                        pl.ds(p * PASSN + sid * CPS + t * C, C)])
            pltpu.sync_copy(deg_sh.at[pl.ds(sid * dzps, dzps)],
                            deg_hbm.at[cid].at[pl.ds(p * DROWSP + sid * dzps,
                                                     dzps)])

            plsc.subcore_barrier()

    return k(nf2, src, dst, w)


_BLK = 1000


def _tc_linear(feat, deg_cols, n_feats, Wt, b2):
    def body(acc0_ref, acc1_ref, deg_ref, nf_ref, wt_ref, b_ref, out_ref):
        deg = jnp.sum(deg_ref[...], axis=1, keepdims=True)
        inv = 1.0 / jnp.maximum(deg, 1.0)
        ha = acc0_ref[...] * inv
        hb = acc1_ref[...] * inv
        wt = wt_ref[...]
        o = jnp.dot(ha, wt[:HALF], preferred_element_type=jnp.float32)
        o = o + jnp.dot(hb, wt[HALF:D], preferred_element_type=jnp.float32)
        o = o + jnp.dot(nf_ref[...], wt[D:], preferred_element_type=jnp.float32)
        out_ref[...] = jnp.maximum(o + b_ref[...], 0.0)

    return pl.pallas_call(
        body,
        grid=(N // _BLK,),
        in_specs=[
            pl.BlockSpec((None, _BLK, HALF), lambda i: (0, i, 0)),
            pl.BlockSpec((None, _BLK, HALF), lambda i: (1, i, 0)),
            pl.BlockSpec((_BLK, 16), lambda i: (i, 0)),
            pl.BlockSpec((_BLK, D), lambda i: (i, 0)),
            pl.BlockSpec((2 * D, D), lambda i: (0, 0)),
            pl.BlockSpec((1, D), lambda i: (0, 0)),
        ],
        out_specs=pl.BlockSpec((_BLK, D), lambda i: (i, 0)),
        out_shape=jax.ShapeDtypeStruct((N, D), jnp.float32),
    )(feat, feat, deg_cols, n_feats, Wt, b2)


def kernel(n_feats, edge_index, e_weights, W, b):
    src = edge_index[0].astype(jnp.int32)
    dst = edge_index[1].astype(jnp.int32)
    w = e_weights.reshape(E).astype(jnp.float32)
    nf2 = n_feats.reshape(N, 2, HALF).transpose(1, 0, 2).reshape(2 * N, HALF)
    feat, deg = _sc_segment_sum(nf2, src, dst, w)
    deg_cols = (deg.reshape(2, NPASS, DROWSP * 128)[:, :, :DNODE * 16]
                .reshape(NPAD, 16))
    return _tc_linear(feat, deg_cols, n_feats, W.T, b.reshape(1, D))


# speedup vs baseline: 3.2963x; 2.0766x over previous
import dataclasses
import functools

import jax
import jax.numpy as jnp
from jax import lax
from jax.experimental import pallas as pl
from jax.experimental.pallas import tpu as pltpu
from jax.experimental.pallas import tpu_sc as plsc

N = 10000
E = 160000
D = 256
HALF = 128
NPAD = 10240
C = 80
NSUB = 16
E2 = 163840
EPS = E2 // NSUB
NC = EPS // C
PKROW = 3 * C
RPS = NPAD // NSUB
NHALF = NPAD // 2
DROWS = NHALF * 16 // 128
DPS = DROWS // NSUB
CD = 128
NCD = EPS // CD


def _compiler_params():
    cp = pltpu.CompilerParams()
    if "needs_layout_passes" in pltpu.CompilerParams.__dataclass_fields__:
        cp = dataclasses.replace(cp, needs_layout_passes=False)
    return cp


def _sc_segment_sum(nf2, pk):
    mesh = plsc.VectorSubcoreMesh(core_axis_name="c", subcore_axis_name="s")

    @functools.partial(
        pl.kernel,
        compiler_params=_compiler_params(),
        out_type=jax.ShapeDtypeStruct((2, NPAD, HALF), jnp.float32),
        mesh=mesh,
        scratch_types=[
            pltpu.VMEM((4, PKROW), jnp.int32),
            pltpu.VMEM((4, C), jnp.int32),
            pltpu.VMEM((4, C), jnp.int32),
            pltpu.VMEM((2, C, HALF), jnp.float32),
            pltpu.VMEM((2, C, HALF), jnp.float32),
            pltpu.VMEM_SHARED((NPAD, HALF), jnp.float32),
            pltpu.SemaphoreType.DMA,
            pltpu.SemaphoreType.DMA,
            pltpu.SemaphoreType.DMA,
            pltpu.SemaphoreType.DMA,
            pltpu.SemaphoreType.DMA,
            pltpu.SemaphoreType.DMA,
        ],
    )
    def k(nf2_hbm, pk_hbm, feat_hbm,
          idx_v, adj_v, rel_v, rows_v, msg_v, acc_sh,
          si0, si1, si2, si3, sg0, sg1):
        cid = lax.axis_index("c")
        sid = lax.axis_index("s")
        zero16 = jnp.zeros((16,), jnp.float32)
        off = cid * N
        row0 = sid * NC
        sem_i = [si0, si1, si2, si3]
        sem_g = [sg0, sg1]

        def load_idx(kk, b):
            pltpu.async_copy(pk_hbm.at[row0 + kk], idx_v.at[b], sem_i[b])

        def wait_idx(kk, b):
            pltpu.make_async_copy(
                pk_hbm.at[row0 + kk], idx_v.at[b], sem_i[b]).wait()

        def prep(b, gb):
            @pl.loop(0, C, step=16)
            def _(i):
                s16 = idx_v[b, pl.ds(i, 16)]
                s16 = jnp.minimum(jnp.maximum(s16, 0), N - 1)
                adj_v[b, pl.ds(i, 16)] = s16 + off
                d16 = idx_v[b, pl.ds(C + i, 16)]
                rel_v[b, pl.ds(i, 16)] = jnp.minimum(
                    jnp.maximum(d16, 0), NPAD - 1)

            pltpu.async_copy(nf2_hbm.at[adj_v.at[b]], rows_v.at[gb],
                             sem_g[gb])

        def wait_gather(b, gb):
            pltpu.make_async_copy(
                nf2_hbm.at[adj_v.at[b]], rows_v.at[gb], sem_g[gb]).wait()

        def finish(b, gb):
            @pl.loop(0, C, step=16)
            def _(i):
                w16 = lax.bitcast_convert_type(
                    idx_v[b, pl.ds(2 * C + i, 16)], jnp.float32)
                for r in range(16):
                    wv = jnp.full((16,), w16[r], jnp.float32)
                    for j in range(HALF // 16):
                        msg_v[gb, i + r, pl.ds(j * 16, 16)] = (
                            rows_v[gb, i + r, pl.ds(j * 16, 16)] * wv)

            pltpu.sync_copy(msg_v.at[gb], acc_sh.at[rel_v.at[b]], add=True)

        @pl.loop(0, C)
        def _(r):
            for j in range(HALF // 16):
                msg_v[0, r, pl.ds(j * 16, 16)] = zero16

        abase = sid * RPS
        for t in range(RPS // C):
            pltpu.sync_copy(msg_v.at[0], acc_sh.at[pl.ds(abase + t * C, C)])

        plsc.subcore_barrier()

        load_idx(0, 0)
        wait_idx(0, 0)
        prep(0, 0)
        load_idx(1, 1)

        @pl.loop(0, NC // 4)
        def _(mm):
            kk = mm * 4
            for j in range(4):
                b, nb = j % 4, (j + 1) % 4
                gb, ngb = j % 2, (j + 1) % 2
                wait_idx(kk + j + 1, nb)
                prep(nb, ngb)
                load_idx(kk + j + 2, (j + 2) % 4)
                wait_gather(b, gb)
                finish(b, gb)

        wait_gather(0, 0)
        wait_idx(NC + 1, 1)

        plsc.subcore_barrier()

        for t in range(RPS // C):
            pltpu.sync_copy(
                acc_sh.at[pl.ds(abase + t * C, C)],
                feat_hbm.at[cid].at[pl.ds(abase + t * C, C)])

    return k(nf2, pk)


def _sc_degree(dstp):
    mesh = plsc.VectorSubcoreMesh(core_axis_name="c", subcore_axis_name="s")

    @functools.partial(
        pl.kernel,
        compiler_params=_compiler_params(),
        out_type=jax.ShapeDtypeStruct((2, DROWS, 128), jnp.float32),
        mesh=mesh,
        scratch_types=[
            pltpu.VMEM((CD,), jnp.int32),
            pltpu.VMEM((DROWS, 128), jnp.float32),
            pltpu.VMEM((DROWS // 128, 128), jnp.int32),
            pltpu.VMEM_SHARED((DROWS, 128), jnp.float32),
        ],
    )
    def k(dst_hbm, deg_hbm, dst_v, degh_v, id_v, deg_sh):
        cid = lax.axis_index("c")
        sid = lax.axis_index("s")
        zero16 = jnp.zeros((16,), jnp.float32)
        one16 = jnp.ones((16,), jnp.float32)
        lane16 = jnp.arange(16, dtype=jnp.int32)
        ebase = sid * EPS
        deg_lo = cid * NHALF

        @pl.loop(0, DROWS)
        def _(r):
            for j in range(128 // 16):
                degh_v[r, pl.ds(j * 16, 16)] = zero16

        for j in range(DROWS // 128):
            for t in range(128 // 16):
                id_v[j, pl.ds(t * 16, 16)] = lane16 + (j * 128 + t * 16)

        pltpu.sync_copy(degh_v.at[pl.ds(0, DPS)],
                        deg_sh.at[pl.ds(sid * DPS, DPS)])

        plsc.subcore_barrier()

        @pl.loop(0, NCD)
        def _(kk):
            pltpu.sync_copy(dst_hbm.at[pl.ds(ebase + kk * CD, CD)], dst_v)

            @pl.loop(0, CD, step=16)
            def _(i):
                d16 = dst_v[pl.ds(i, 16)]
                rel = d16 - deg_lo
                mask = (rel >= 0) & (rel < NHALF)
                relc = jnp.where(mask, rel, 0)
                flat = relc * 16 + lane16
                plsc.addupdate_scatter(
                    degh_v, [flat >> 7, flat & 127], one16, mask=mask)

        for j in range(DROWS // 128):
            pltpu.sync_copy(degh_v.at[pl.ds(j * 128, 128)],
                            deg_sh.at[id_v.at[j]], add=True)

        plsc.subcore_barrier()

        pltpu.sync_copy(deg_sh.at[pl.ds(sid * DPS, DPS)],
                        deg_hbm.at[cid].at[pl.ds(sid * DPS, DPS)])

    return k(dstp)


_BLK = 1000


def _tc_linear(feat, deg_cols, n_feats, Wt, b2):
    def body(acc0_ref, acc1_ref, deg_ref, nf_ref, wt_ref, b_ref, out_ref):
        deg = jnp.sum(deg_ref[...], axis=1, keepdims=True)
        inv = 1.0 / jnp.maximum(deg, 1.0)
        ha = acc0_ref[...] * inv
        hb = acc1_ref[...] * inv
        wt = wt_ref[...]
        o = jnp.dot(ha, wt[:HALF], preferred_element_type=jnp.float32)
        o = o + jnp.dot(hb, wt[HALF:D], preferred_element_type=jnp.float32)
        o = o + jnp.dot(nf_ref[...], wt[D:], preferred_element_type=jnp.float32)
        out_ref[...] = jnp.maximum(o + b_ref[...], 0.0)

    return pl.pallas_call(
        body,
        grid=(N // _BLK,),
        in_specs=[
            pl.BlockSpec((None, _BLK, HALF), lambda i: (0, i, 0)),
            pl.BlockSpec((None, _BLK, HALF), lambda i: (1, i, 0)),
            pl.BlockSpec((_BLK, 16), lambda i: (i, 0)),
            pl.BlockSpec((_BLK, D), lambda i: (i, 0)),
            pl.BlockSpec((2 * D, D), lambda i: (0, 0)),
            pl.BlockSpec((1, D), lambda i: (0, 0)),
        ],
        out_specs=pl.BlockSpec((_BLK, D), lambda i: (i, 0)),
        out_shape=jax.ShapeDtypeStruct((N, D), jnp.float32),
    )(feat, feat, deg_cols, n_feats, Wt, b2)


def kernel(n_feats, edge_index, e_weights, W, b):
    src = edge_index[0].astype(jnp.int32)
    dst = edge_index[1].astype(jnp.int32)
    w = e_weights.reshape(E).astype(jnp.float32)
    npd = E2 - E
    srcp = jnp.concatenate([src, jnp.zeros((npd,), jnp.int32)])
    dstp = jnp.concatenate([dst, jnp.full((npd,), NPAD - 1, jnp.int32)])
    wp = jnp.concatenate([w, jnp.zeros((npd,), jnp.float32)])
    pk = jnp.concatenate(
        [srcp.reshape(E2 // C, C),
         dstp.reshape(E2 // C, C),
         lax.bitcast_convert_type(wp, jnp.int32).reshape(E2 // C, C)],
        axis=1)
    pk = jnp.pad(pk, ((0, 4), (0, 0)))
    nf2 = n_feats.reshape(N, 2, HALF).transpose(1, 0, 2).reshape(2 * N, HALF)
    feat = _sc_segment_sum(nf2, pk)
    deg = _sc_degree(dstp)
    deg_cols = deg.reshape(NPAD, 16)
    return _tc_linear(feat, deg_cols, n_feats, W.T, b.reshape(1, D))


# speedup vs baseline: 3.4617x; 1.0502x over previous
import dataclasses
import functools

import jax
import jax.numpy as jnp
from jax import lax
from jax.experimental import pallas as pl
from jax.experimental.pallas import tpu as pltpu
from jax.experimental.pallas import tpu_sc as plsc

N = 10000
E = 160000
D = 256
HALF = 128
NPAD = 10240
C = 80
NSUB = 16
E2 = 163840
EPS = E2 // NSUB
NC = EPS // C
PKROW = 3 * C
RPS = NPAD // NSUB
NHALF = NPAD // 2
DROWS = NHALF * 16 // 128
DPS = DROWS // NSUB
CD = 128
NCD = EPS // CD


def _compiler_params():
    cp = pltpu.CompilerParams()
    if "needs_layout_passes" in pltpu.CompilerParams.__dataclass_fields__:
        cp = dataclasses.replace(cp, needs_layout_passes=False)
    return cp


def _sc_segment_sum(nf2, pk):
    mesh = plsc.VectorSubcoreMesh(core_axis_name="c", subcore_axis_name="s")

    @functools.partial(
        pl.kernel,
        compiler_params=_compiler_params(),
        out_type=jax.ShapeDtypeStruct((2, NPAD, HALF), jnp.float32),
        mesh=mesh,
        scratch_types=[
            pltpu.VMEM((4, PKROW), jnp.int32),
            pltpu.VMEM((4, C), jnp.int32),
            pltpu.VMEM((4, C), jnp.int32),
            pltpu.VMEM((2, C, HALF), jnp.float32),
            pltpu.VMEM((2, C, HALF), jnp.float32),
            pltpu.VMEM_SHARED((NPAD, HALF), jnp.float32),
            pltpu.SemaphoreType.DMA,
            pltpu.SemaphoreType.DMA,
            pltpu.SemaphoreType.DMA,
            pltpu.SemaphoreType.DMA,
            pltpu.SemaphoreType.DMA,
            pltpu.SemaphoreType.DMA,
            pltpu.SemaphoreType.DMA,
            pltpu.SemaphoreType.DMA,
        ],
    )
    def k(nf2_hbm, pk_hbm, feat_hbm,
          idx_v, adj_v, rel_v, rows_v, msg_v, acc_sh,
          si0, si1, si2, si3, sg0, sg1, ss0, ss1):
        cid = lax.axis_index("c")
        sid = lax.axis_index("s")
        zero16 = jnp.zeros((16,), jnp.float32)
        off = cid * N
        row0 = sid * NC
        sem_i = [si0, si1, si2, si3]
        sem_g = [sg0, sg1]
        sem_s = [ss0, ss1]

        def load_idx(kk, b):
            pltpu.async_copy(pk_hbm.at[row0 + kk], idx_v.at[b], sem_i[b])

        def wait_idx(kk, b):
            pltpu.make_async_copy(
                pk_hbm.at[row0 + kk], idx_v.at[b], sem_i[b]).wait()

        def prep(b, gb):
            @pl.loop(0, C, step=16)
            def _(i):
                s16 = idx_v[b, pl.ds(i, 16)]
                s16 = jnp.minimum(jnp.maximum(s16, 0), N - 1)
                adj_v[b, pl.ds(i, 16)] = s16 + off
                d16 = idx_v[b, pl.ds(C + i, 16)]
                rel_v[b, pl.ds(i, 16)] = jnp.minimum(
                    jnp.maximum(d16, 0), NPAD - 1)

            pltpu.async_copy(nf2_hbm.at[adj_v.at[b]], rows_v.at[gb],
                             sem_g[gb])

        def wait_gather(b, gb):
            pltpu.make_async_copy(
                nf2_hbm.at[adj_v.at[b]], rows_v.at[gb], sem_g[gb]).wait()

        def drain_scatter(b, gb):
            pltpu.make_async_copy(
                msg_v.at[gb], acc_sh.at[rel_v.at[b]], sem_s[gb]).wait()

        def finish(b, gb):
            @pl.loop(0, C, step=16)
            def _(i):
                w16 = lax.bitcast_convert_type(
                    idx_v[b, pl.ds(2 * C + i, 16)], jnp.float32)
                for r in range(16):
                    wv = jnp.full((16,), w16[r], jnp.float32)
                    for j in range(HALF // 16):
                        msg_v[gb, i + r, pl.ds(j * 16, 16)] = (
                            rows_v[gb, i + r, pl.ds(j * 16, 16)] * wv)

            pltpu.async_copy(msg_v.at[gb], acc_sh.at[rel_v.at[b]],
                             sem_s[gb], add=True)

        @pl.loop(0, C)
        def _(r):
            for g in range(2):
                for j in range(HALF // 16):
                    msg_v[g, r, pl.ds(j * 16, 16)] = zero16

        zi16 = jnp.zeros((16,), jnp.int32)

        @pl.loop(0, C, step=16)
        def _(i):
            rel_v[2, pl.ds(i, 16)] = zi16
            rel_v[3, pl.ds(i, 16)] = zi16

        abase = sid * RPS
        for t in range(RPS // C):
            pltpu.sync_copy(msg_v.at[0], acc_sh.at[pl.ds(abase + t * C, C)])

        plsc.subcore_barrier()

        pltpu.async_copy(msg_v.at[0], acc_sh.at[rel_v.at[2]], sem_s[0],
                         add=True)
        pltpu.async_copy(msg_v.at[1], acc_sh.at[rel_v.at[3]], sem_s[1],
                         add=True)

        load_idx(0, 0)
        wait_idx(0, 0)
        prep(0, 0)
        load_idx(1, 1)

        @pl.loop(0, NC // 4)
        def _(mm):
            kk = mm * 4
            for j in range(4):
                b, nb = j % 4, (j + 1) % 4
                gb, ngb = j % 2, (j + 1) % 2
                wait_idx(kk + j + 1, nb)
                prep(nb, ngb)
                load_idx(kk + j + 2, (j + 2) % 4)
                wait_gather(b, gb)
                drain_scatter((j + 2) % 4, gb)
                finish(b, gb)

        drain_scatter(2, 0)
        drain_scatter(3, 1)
        wait_gather(0, 0)
        wait_idx(NC + 1, 1)

        plsc.subcore_barrier()

        for t in range(RPS // C):
            pltpu.sync_copy(
                acc_sh.at[pl.ds(abase + t * C, C)],
                feat_hbm.at[cid].at[pl.ds(abase + t * C, C)])

    return k(nf2, pk)


def _sc_degree(dstp):
    mesh = plsc.VectorSubcoreMesh(core_axis_name="c", subcore_axis_name="s")

    @functools.partial(
        pl.kernel,
        compiler_params=_compiler_params(),
        out_type=jax.ShapeDtypeStruct((2, DROWS, 128), jnp.float32),
        mesh=mesh,
        scratch_types=[
            pltpu.VMEM((CD,), jnp.int32),
            pltpu.VMEM((DROWS, 128), jnp.float32),
            pltpu.VMEM((DROWS // 128, 128), jnp.int32),
            pltpu.VMEM_SHARED((DROWS, 128), jnp.float32),
        ],
    )
    def k(dst_hbm, deg_hbm, dst_v, degh_v, id_v, deg_sh):
        cid = lax.axis_index("c")
        sid = lax.axis_index("s")
        zero16 = jnp.zeros((16,), jnp.float32)
        one16 = jnp.ones((16,), jnp.float32)
        lane16 = jnp.arange(16, dtype=jnp.int32)
        ebase = sid * EPS
        deg_lo = cid * NHALF

        @pl.loop(0, DROWS)
        def _(r):
            for j in range(128 // 16):
                degh_v[r, pl.ds(j * 16, 16)] = zero16

        for j in range(DROWS // 128):
            for t in range(128 // 16):
                id_v[j, pl.ds(t * 16, 16)] = lane16 + (j * 128 + t * 16)

        pltpu.sync_copy(degh_v.at[pl.ds(0, DPS)],
                        deg_sh.at[pl.ds(sid * DPS, DPS)])

        plsc.subcore_barrier()

        @pl.loop(0, NCD)
        def _(kk):
            pltpu.sync_copy(dst_hbm.at[pl.ds(ebase + kk * CD, CD)], dst_v)

            @pl.loop(0, CD, step=16)
            def _(i):
                d16 = dst_v[pl.ds(i, 16)]
                rel = d16 - deg_lo
                mask = (rel >= 0) & (rel < NHALF)
                relc = jnp.where(mask, rel, 0)
                flat = relc * 16 + lane16
                plsc.addupdate_scatter(
                    degh_v, [flat >> 7, flat & 127], one16, mask=mask)

        for j in range(DROWS // 128):
            pltpu.sync_copy(degh_v.at[pl.ds(j * 128, 128)],
                            deg_sh.at[id_v.at[j]], add=True)

        plsc.subcore_barrier()

        pltpu.sync_copy(deg_sh.at[pl.ds(sid * DPS, DPS)],
                        deg_hbm.at[cid].at[pl.ds(sid * DPS, DPS)])

    return k(dstp)


_BLK = 1000


def _tc_linear(feat, deg_cols, n_feats, Wt, b2):
    def body(acc0_ref, acc1_ref, deg_ref, nf_ref, wt_ref, b_ref, out_ref):
        deg = jnp.sum(deg_ref[...], axis=1, keepdims=True)
        inv = 1.0 / jnp.maximum(deg, 1.0)
        ha = acc0_ref[...] * inv
        hb = acc1_ref[...] * inv
        wt = wt_ref[...]
        o = jnp.dot(ha, wt[:HALF], preferred_element_type=jnp.float32)
        o = o + jnp.dot(hb, wt[HALF:D], preferred_element_type=jnp.float32)
        o = o + jnp.dot(nf_ref[...], wt[D:], preferred_element_type=jnp.float32)
        out_ref[...] = jnp.maximum(o + b_ref[...], 0.0)

    return pl.pallas_call(
        body,
        grid=(N // _BLK,),
        in_specs=[
            pl.BlockSpec((None, _BLK, HALF), lambda i: (0, i, 0)),
            pl.BlockSpec((None, _BLK, HALF), lambda i: (1, i, 0)),
            pl.BlockSpec((_BLK, 16), lambda i: (i, 0)),
            pl.BlockSpec((_BLK, D), lambda i: (i, 0)),
            pl.BlockSpec((2 * D, D), lambda i: (0, 0)),
            pl.BlockSpec((1, D), lambda i: (0, 0)),
        ],
        out_specs=pl.BlockSpec((_BLK, D), lambda i: (i, 0)),
        out_shape=jax.ShapeDtypeStruct((N, D), jnp.float32),
    )(feat, feat, deg_cols, n_feats, Wt, b2)


def kernel(n_feats, edge_index, e_weights, W, b):
    src = edge_index[0].astype(jnp.int32)
    dst = edge_index[1].astype(jnp.int32)
    w = e_weights.reshape(E).astype(jnp.float32)
    npd = E2 - E
    srcp = jnp.concatenate([src, jnp.zeros((npd,), jnp.int32)])
    dstp = jnp.concatenate([dst, jnp.full((npd,), NPAD - 1, jnp.int32)])
    wp = jnp.concatenate([w, jnp.zeros((npd,), jnp.float32)])
    pk = jnp.concatenate(
        [srcp.reshape(E2 // C, C),
         dstp.reshape(E2 // C, C),
         lax.bitcast_convert_type(wp, jnp.int32).reshape(E2 // C, C)],
        axis=1)
    pk = jnp.pad(pk, ((0, 4), (0, 0)))
    nf2 = n_feats.reshape(N, 2, HALF).transpose(1, 0, 2).reshape(2 * N, HALF)
    feat = _sc_segment_sum(nf2, pk)
    deg = _sc_degree(dstp)
    deg_cols = deg.reshape(NPAD, 16)
    return _tc_linear(feat, deg_cols, n_feats, W.T, b.reshape(1, D))


# speedup vs baseline: 3.6295x; 1.0485x over previous
import dataclasses
import functools

import jax
import jax.numpy as jnp
from jax import lax
from jax.experimental import pallas as pl
from jax.experimental.pallas import tpu as pltpu
from jax.experimental.pallas import tpu_sc as plsc

N = 10000
E = 160000
D = 256
HALF = 128
NPAD = 10240
C = 80
NSUB = 16
E2 = 163840
EPS = E2 // NSUB
NC = EPS // C
PKROW = 3 * C
RPS = NPAD // NSUB
NHALF = NPAD // 2
DROWS = NHALF * 16 // 128
DPS = DROWS // NSUB
CD = 128
NCD = EPS // CD


def _compiler_params():
    cp = pltpu.CompilerParams()
    if "needs_layout_passes" in pltpu.CompilerParams.__dataclass_fields__:
        cp = dataclasses.replace(cp, needs_layout_passes=False)
    return cp


def _sc_segment_sum(nf2, pk):
    mesh = plsc.VectorSubcoreMesh(core_axis_name="c", subcore_axis_name="s")

    @functools.partial(
        pl.kernel,
        compiler_params=_compiler_params(),
        out_type=jax.ShapeDtypeStruct((2, NPAD, HALF), jnp.float32),
        mesh=mesh,
        scratch_types=[
            pltpu.VMEM((4, PKROW), jnp.int32),
            pltpu.VMEM((4, C), jnp.int32),
            pltpu.VMEM((4, C), jnp.int32),
            pltpu.VMEM((2, C, HALF), jnp.float32),
            pltpu.VMEM((2, C, HALF), jnp.float32),
            pltpu.VMEM_SHARED((NPAD, HALF), jnp.float32),
            pltpu.SemaphoreType.DMA,
            pltpu.SemaphoreType.DMA,
            pltpu.SemaphoreType.DMA,
            pltpu.SemaphoreType.DMA,
            pltpu.SemaphoreType.DMA,
            pltpu.SemaphoreType.DMA,
            pltpu.SemaphoreType.DMA,
            pltpu.SemaphoreType.DMA,
        ],
    )
    def k(nf2_hbm, pk_hbm, feat_hbm,
          idx_v, adj_v, rel_v, rows_v, msg_v, acc_sh,
          si0, si1, si2, si3, sg0, sg1, ss0, ss1):
        cid = lax.axis_index("c")
        sid = lax.axis_index("s")
        zero16 = jnp.zeros((16,), jnp.float32)
        off = cid * N
        row0 = sid * NC
        sem_i = [si0, si1, si2, si3]
        sem_g = [sg0, sg1]
        sem_s = [ss0, ss1]

        def load_idx(kk, b):
            pltpu.async_copy(pk_hbm.at[row0 + kk], idx_v.at[b], sem_i[b])

        def wait_idx(kk, b):
            pltpu.make_async_copy(
                pk_hbm.at[row0 + kk], idx_v.at[b], sem_i[b]).wait()

        def prep(b, gb):
            @pl.loop(0, C, step=16)
            def _(i):
                s16 = idx_v[b, pl.ds(i, 16)]
                s16 = jnp.minimum(jnp.maximum(s16, 0), N - 1)
                adj_v[b, pl.ds(i, 16)] = s16 + off
                d16 = idx_v[b, pl.ds(C + i, 16)]
                rel_v[b, pl.ds(i, 16)] = jnp.minimum(
                    jnp.maximum(d16, 0), NPAD - 1)

            pltpu.async_copy(nf2_hbm.at[adj_v.at[b]], rows_v.at[gb],
                             sem_g[gb])

        def wait_gather(b, gb):
            pltpu.make_async_copy(
                nf2_hbm.at[adj_v.at[b]], rows_v.at[gb], sem_g[gb]).wait()

        def drain_scatter(b, gb):
            pltpu.make_async_copy(
                msg_v.at[gb], acc_sh.at[rel_v.at[b]], sem_s[gb]).wait()

        def finish(b, gb):
            @pl.loop(0, C, step=16)
            def _(i):
                w16 = lax.bitcast_convert_type(
                    idx_v[b, pl.ds(2 * C + i, 16)], jnp.float32)
                for r in range(16):
                    wv = jnp.full((16,), w16[r], jnp.float32)
                    for j in range(HALF // 16):
                        msg_v[gb, i + r, pl.ds(j * 16, 16)] = (
                            rows_v[gb, i + r, pl.ds(j * 16, 16)] * wv)

            pltpu.async_copy(msg_v.at[gb], acc_sh.at[rel_v.at[b]],
                             sem_s[gb], add=True)

        @pl.loop(0, C)
        def _(r):
            for g in range(2):
                for j in range(HALF // 16):
                    msg_v[g, r, pl.ds(j * 16, 16)] = zero16

        zi16 = jnp.zeros((16,), jnp.int32)

        @pl.loop(0, C, step=16)
        def _(i):
            rel_v[2, pl.ds(i, 16)] = zi16
            rel_v[3, pl.ds(i, 16)] = zi16

        abase = sid * RPS
        for t in range(RPS // C):
            pltpu.sync_copy(msg_v.at[0], acc_sh.at[pl.ds(abase + t * C, C)])

        plsc.subcore_barrier()

        pltpu.async_copy(msg_v.at[0], acc_sh.at[rel_v.at[2]], sem_s[0],
                         add=True)
        pltpu.async_copy(msg_v.at[1], acc_sh.at[rel_v.at[3]], sem_s[1],
                         add=True)

        load_idx(0, 0)
        wait_idx(0, 0)
        prep(0, 0)
        load_idx(1, 1)

        @pl.loop(0, NC // 4)
        def _(mm):
            kk = mm * 4
            for j in range(4):
                b, nb = j % 4, (j + 1) % 4
                gb, ngb = j % 2, (j + 1) % 2
                wait_idx(kk + j + 1, nb)
                prep(nb, ngb)
                load_idx(kk + j + 2, (j + 2) % 4)
                wait_gather(b, gb)
                drain_scatter((j + 2) % 4, gb)
                finish(b, gb)

        drain_scatter(2, 0)
        drain_scatter(3, 1)
        wait_gather(0, 0)
        wait_idx(NC + 1, 1)

        plsc.subcore_barrier()

        for t in range(RPS // C):
            pltpu.sync_copy(
                acc_sh.at[pl.ds(abase + t * C, C)],
                feat_hbm.at[cid].at[pl.ds(abase + t * C, C)])

    return k(nf2, pk)


def _sc_degree(dstp):
    mesh = plsc.VectorSubcoreMesh(core_axis_name="c", subcore_axis_name="s")

    @functools.partial(
        pl.kernel,
        compiler_params=_compiler_params(),
        out_type=jax.ShapeDtypeStruct((2, DROWS, 128), jnp.float32),
        mesh=mesh,
        scratch_types=[
            pltpu.VMEM((2, CD), jnp.int32),
            pltpu.VMEM((DROWS, 128), jnp.float32),
            pltpu.VMEM((DROWS // 128, 128), jnp.int32),
            pltpu.VMEM_SHARED((DROWS, 128), jnp.float32),
            pltpu.SemaphoreType.DMA,
            pltpu.SemaphoreType.DMA,
        ],
    )
    def k(dst_hbm, deg_hbm, dst_v, degh_v, id_v, deg_sh, sd0, sd1):
        cid = lax.axis_index("c")
        sid = lax.axis_index("s")
        zero16 = jnp.zeros((16,), jnp.float32)
        one16 = jnp.ones((16,), jnp.float32)
        lane16 = jnp.arange(16, dtype=jnp.int32)
        ebase = sid * EPS
        deg_lo = cid * NHALF

        @pl.loop(0, DROWS)
        def _(r):
            for j in range(128 // 16):
                degh_v[r, pl.ds(j * 16, 16)] = zero16

        for j in range(DROWS // 128):
            for t in range(128 // 16):
                id_v[j, pl.ds(t * 16, 16)] = lane16 + (j * 128 + t * 16)

        pltpu.sync_copy(degh_v.at[pl.ds(0, DPS)],
                        deg_sh.at[pl.ds(sid * DPS, DPS)])

        plsc.subcore_barrier()

        semd = [sd0, sd1]

        def load_dst(kk, bb):
            pltpu.async_copy(dst_hbm.at[pl.ds(ebase + kk * CD, CD)],
                             dst_v.at[bb], semd[bb])

        def wait_dst(kk, bb):
            pltpu.make_async_copy(dst_hbm.at[pl.ds(ebase + kk * CD, CD)],
                                  dst_v.at[bb], semd[bb]).wait()

        load_dst(0, 0)

        @pl.loop(0, NCD // 2)
        def _(mm):
            kk = mm * 2
            for j in range(2):
                bb = j
                load_dst(kk + j + 1, 1 - j)
                wait_dst(kk + j, bb)

                @pl.loop(0, CD, step=16)
                def _(i):
                    d16 = dst_v[bb, pl.ds(i, 16)]
                    rel = d16 - deg_lo
                    mask = (rel >= 0) & (rel < NHALF)
                    relc = jnp.where(mask, rel, 0)
                    flat = relc * 16 + lane16
                    plsc.addupdate_scatter(
                        degh_v, [flat >> 7, flat & 127], one16, mask=mask)

        wait_dst(NCD, 0)

        for j in range(DROWS // 128):
            pltpu.sync_copy(degh_v.at[pl.ds(j * 128, 128)],
                            deg_sh.at[id_v.at[j]], add=True)

        plsc.subcore_barrier()

        pltpu.sync_copy(deg_sh.at[pl.ds(sid * DPS, DPS)],
                        deg_hbm.at[cid].at[pl.ds(sid * DPS, DPS)])

    return k(dstp)


_BLK = 1000


def _tc_linear(feat, deg_cols, n_feats, Wt, b2):
    def body(acc0_ref, acc1_ref, deg_ref, nf_ref, wt_ref, b_ref, out_ref):
        deg = jnp.sum(deg_ref[...], axis=1, keepdims=True)
        inv = 1.0 / jnp.maximum(deg, 1.0)
        ha = acc0_ref[...] * inv
        hb = acc1_ref[...] * inv
        wt = wt_ref[...]
        o = jnp.dot(ha, wt[:HALF], preferred_element_type=jnp.float32)
        o = o + jnp.dot(hb, wt[HALF:D], preferred_element_type=jnp.float32)
        o = o + jnp.dot(nf_ref[...], wt[D:], preferred_element_type=jnp.float32)
        out_ref[...] = jnp.maximum(o + b_ref[...], 0.0)

    return pl.pallas_call(
        body,
        grid=(N // _BLK,),
        in_specs=[
            pl.BlockSpec((None, _BLK, HALF), lambda i: (0, i, 0)),
            pl.BlockSpec((None, _BLK, HALF), lambda i: (1, i, 0)),
            pl.BlockSpec((_BLK, 16), lambda i: (i, 0)),
            pl.BlockSpec((_BLK, D), lambda i: (i, 0)),
            pl.BlockSpec((2 * D, D), lambda i: (0, 0)),
            pl.BlockSpec((1, D), lambda i: (0, 0)),
        ],
        out_specs=pl.BlockSpec((_BLK, D), lambda i: (i, 0)),
        out_shape=jax.ShapeDtypeStruct((N, D), jnp.float32),
    )(feat, feat, deg_cols, n_feats, Wt, b2)


def kernel(n_feats, edge_index, e_weights, W, b):
    src = edge_index[0].astype(jnp.int32)
    dst = edge_index[1].astype(jnp.int32)
    w = e_weights.reshape(E).astype(jnp.float32)
    npd = E2 - E
    srcp = jnp.concatenate([src, jnp.zeros((npd,), jnp.int32)])
    dstp = jnp.concatenate([dst, jnp.full((npd,), NPAD - 1, jnp.int32)])
    wp = jnp.concatenate([w, jnp.zeros((npd,), jnp.float32)])
    pk = jnp.concatenate(
        [srcp.reshape(E2 // C, C),
         dstp.reshape(E2 // C, C),
         lax.bitcast_convert_type(wp, jnp.int32).reshape(E2 // C, C)],
        axis=1)
    pk = jnp.pad(pk, ((0, 4), (0, 0)))
    nf2 = n_feats.reshape(N, 2, HALF).transpose(1, 0, 2).reshape(2 * N, HALF)
    feat = _sc_segment_sum(nf2, pk)
    deg = _sc_degree(jnp.pad(dstp, (0, CD)))
    deg_cols = deg.reshape(NPAD, 16)
    return _tc_linear(feat, deg_cols, n_feats, W.T, b.reshape(1, D))


# speedup vs baseline: 3.7637x; 1.0370x over previous
import dataclasses
import functools

import jax
import jax.numpy as jnp
from jax import lax
from jax.experimental import pallas as pl
from jax.experimental.pallas import tpu as pltpu
from jax.experimental.pallas import tpu_sc as plsc

N = 10000
E = 160000
D = 256
HALF = 128
NPAD = 10240
C = 80
NSUB = 16
E2 = 163840
EPS = E2 // NSUB
NC = EPS // C
PKROW = 3 * C
RPS = NPAD // NSUB
NHALF = NPAD // 2
DROWS = NHALF * 16 // 128
DPS = DROWS // NSUB
CD = 128
NCD = EPS // CD


def _compiler_params():
    cp = pltpu.CompilerParams()
    if "needs_layout_passes" in pltpu.CompilerParams.__dataclass_fields__:
        cp = dataclasses.replace(cp, needs_layout_passes=False)
    return cp


def _sc_segment_sum(nf2, pk):
    mesh = plsc.VectorSubcoreMesh(core_axis_name="c", subcore_axis_name="s")

    @functools.partial(
        pl.kernel,
        compiler_params=_compiler_params(),
        out_type=jax.ShapeDtypeStruct((2, NPAD, HALF), jnp.float32),
        mesh=mesh,
        scratch_types=[
            pltpu.VMEM((4, PKROW), jnp.int32),
            pltpu.VMEM((4, C), jnp.int32),
            pltpu.VMEM((4, C), jnp.int32),
            pltpu.VMEM((2, C, HALF), jnp.float32),
            pltpu.VMEM((2, C, HALF), jnp.float32),
            pltpu.VMEM_SHARED((NPAD, HALF), jnp.float32),
            pltpu.SemaphoreType.DMA,
            pltpu.SemaphoreType.DMA,
            pltpu.SemaphoreType.DMA,
            pltpu.SemaphoreType.DMA,
            pltpu.SemaphoreType.DMA,
            pltpu.SemaphoreType.DMA,
            pltpu.SemaphoreType.DMA,
            pltpu.SemaphoreType.DMA,
        ],
    )
    def k(nf2_hbm, pk_hbm, feat_hbm,
          idx_v, adj_v, rel_v, rows_v, msg_v, acc_sh,
          si0, si1, si2, si3, sg0, sg1, ss0, ss1):
        cid = lax.axis_index("c")
        sid = lax.axis_index("s")
        zero16 = jnp.zeros((16,), jnp.float32)
        off = cid * N
        row0 = sid * NC
        sem_i = [si0, si1, si2, si3]
        sem_g = [sg0, sg1]
        sem_s = [ss0, ss1]

        def load_idx(kk, b):
            pltpu.async_copy(pk_hbm.at[row0 + kk], idx_v.at[b], sem_i[b])

        def wait_idx(kk, b):
            pltpu.make_async_copy(
                pk_hbm.at[row0 + kk], idx_v.at[b], sem_i[b]).wait()

        def prep(b, gb):
            @pl.loop(0, C, step=16)
            def _(i):
                s16 = idx_v[b, pl.ds(i, 16)]
                s16 = jnp.minimum(jnp.maximum(s16, 0), N - 1)
                adj_v[b, pl.ds(i, 16)] = s16 + off
                d16 = idx_v[b, pl.ds(C + i, 16)]
                rel_v[b, pl.ds(i, 16)] = jnp.minimum(
                    jnp.maximum(d16, 0), NPAD - 1)

            pltpu.async_copy(nf2_hbm.at[adj_v.at[b]], rows_v.at[gb],
                             sem_g[gb])

        def wait_gather(b, gb):
            pltpu.make_async_copy(
                nf2_hbm.at[adj_v.at[b]], rows_v.at[gb], sem_g[gb]).wait()

        def drain_scatter(b, gb):
            pltpu.make_async_copy(
                msg_v.at[gb], acc_sh.at[rel_v.at[b]], sem_s[gb]).wait()

        def finish(b, gb):
            @pl.loop(0, C, step=16)
            def _(i):
                w16 = lax.bitcast_convert_type(
                    idx_v[b, pl.ds(2 * C + i, 16)], jnp.float32)
                for r in range(16):
                    wv = jnp.full((16,), w16[r], jnp.float32)
                    for j in range(HALF // 16):
                        msg_v[gb, i + r, pl.ds(j * 16, 16)] = (
                            rows_v[gb, i + r, pl.ds(j * 16, 16)] * wv)

            pltpu.async_copy(msg_v.at[gb], acc_sh.at[rel_v.at[b]],
                             sem_s[gb], add=True)

        @pl.loop(0, C)
        def _(r):
            for g in range(2):
                for j in range(HALF // 16):
                    msg_v[g, r, pl.ds(j * 16, 16)] = zero16

        zi16 = jnp.zeros((16,), jnp.int32)

        @pl.loop(0, C, step=16)
        def _(i):
            rel_v[2, pl.ds(i, 16)] = zi16
            rel_v[3, pl.ds(i, 16)] = zi16

        abase = sid * RPS
        for t in range(RPS // C):
            pltpu.sync_copy(msg_v.at[0], acc_sh.at[pl.ds(abase + t * C, C)])

        plsc.subcore_barrier()

        pltpu.async_copy(msg_v.at[0], acc_sh.at[rel_v.at[2]], sem_s[0],
                         add=True)
        pltpu.async_copy(msg_v.at[1], acc_sh.at[rel_v.at[3]], sem_s[1],
                         add=True)

        load_idx(0, 0)
        wait_idx(0, 0)
        prep(0, 0)
        load_idx(1, 1)

        @pl.loop(0, NC // 4)
        def _(mm):
            kk = mm * 4
            for j in range(4):
                b, nb = j % 4, (j + 1) % 4
                gb, ngb = j % 2, (j + 1) % 2
                wait_idx(kk + j + 1, nb)
                prep(nb, ngb)
                load_idx(kk + j + 2, (j + 2) % 4)
                wait_gather(b, gb)
                drain_scatter((j + 2) % 4, gb)
                finish(b, gb)

        drain_scatter(2, 0)
        drain_scatter(3, 1)
        wait_gather(0, 0)
        wait_idx(NC + 1, 1)

        plsc.subcore_barrier()

        for t in range(RPS // C):
            pltpu.sync_copy(
                acc_sh.at[pl.ds(abase + t * C, C)],
                feat_hbm.at[cid].at[pl.ds(abase + t * C, C)])

    return k(nf2, pk)


def _sc_degree(dstp):
    mesh = plsc.VectorSubcoreMesh(core_axis_name="c", subcore_axis_name="s")

    @functools.partial(
        pl.kernel,
        compiler_params=_compiler_params(),
        out_type=jax.ShapeDtypeStruct((2, DROWS, 128), jnp.float32),
        mesh=mesh,
        scratch_types=[
            pltpu.VMEM((EPS,), jnp.int32),
            pltpu.VMEM((DROWS, 128), jnp.float32),
            pltpu.VMEM((DROWS // 128, 128), jnp.int32),
            pltpu.VMEM_SHARED((DROWS, 128), jnp.float32),
        ],
    )
    def k(dst_hbm, deg_hbm, dst_v, degh_v, id_v, deg_sh):
        cid = lax.axis_index("c")
        sid = lax.axis_index("s")
        zero16 = jnp.zeros((16,), jnp.float32)
        one16 = jnp.ones((16,), jnp.float32)
        lane16 = jnp.arange(16, dtype=jnp.int32)
        ebase = sid * EPS
        deg_lo = cid * NHALF

        @pl.loop(0, DROWS)
        def _(r):
            for j in range(128 // 16):
                degh_v[r, pl.ds(j * 16, 16)] = zero16

        for j in range(DROWS // 128):
            for t in range(128 // 16):
                id_v[j, pl.ds(t * 16, 16)] = lane16 + (j * 128 + t * 16)

        pltpu.sync_copy(degh_v.at[pl.ds(0, DPS)],
                        deg_sh.at[pl.ds(sid * DPS, DPS)])

        plsc.subcore_barrier()

        pltpu.sync_copy(dst_hbm.at[pl.ds(ebase, EPS)], dst_v)

        @pl.loop(0, EPS, step=16)
        def _(i):
            d16 = dst_v[pl.ds(i, 16)]
            rel = d16 - deg_lo
            mask = (rel >= 0) & (rel < NHALF)
            relc = jnp.where(mask, rel, 0)
            flat = relc * 16 + lane16
            plsc.addupdate_scatter(
                degh_v, [flat >> 7, flat & 127], one16, mask=mask)

        for j in range(DROWS // 128):
            pltpu.sync_copy(degh_v.at[pl.ds(j * 128, 128)],
                            deg_sh.at[id_v.at[j]], add=True)

        plsc.subcore_barrier()

        pltpu.sync_copy(deg_sh.at[pl.ds(sid * DPS, DPS)],
                        deg_hbm.at[cid].at[pl.ds(sid * DPS, DPS)])

    return k(dstp)


_BLK = 1000


def _tc_linear(feat, deg_cols, n_feats, Wt, b2):
    def body(acc0_ref, acc1_ref, deg_ref, nf_ref, wt_ref, b_ref, out_ref):
        deg = jnp.sum(deg_ref[...], axis=1, keepdims=True)
        inv = 1.0 / jnp.maximum(deg, 1.0)
        ha = acc0_ref[...] * inv
        hb = acc1_ref[...] * inv
        wt = wt_ref[...]
        o = jnp.dot(ha, wt[:HALF], preferred_element_type=jnp.float32)
        o = o + jnp.dot(hb, wt[HALF:D], preferred_element_type=jnp.float32)
        o = o + jnp.dot(nf_ref[...], wt[D:], preferred_element_type=jnp.float32)
        out_ref[...] = jnp.maximum(o + b_ref[...], 0.0)

    return pl.pallas_call(
        body,
        grid=(N // _BLK,),
        in_specs=[
            pl.BlockSpec((None, _BLK, HALF), lambda i: (0, i, 0)),
            pl.BlockSpec((None, _BLK, HALF), lambda i: (1, i, 0)),
            pl.BlockSpec((_BLK, 16), lambda i: (i, 0)),
            pl.BlockSpec((_BLK, D), lambda i: (i, 0)),
            pl.BlockSpec((2 * D, D), lambda i: (0, 0)),
            pl.BlockSpec((1, D), lambda i: (0, 0)),
        ],
        out_specs=pl.BlockSpec((_BLK, D), lambda i: (i, 0)),
        out_shape=jax.ShapeDtypeStruct((N, D), jnp.float32),
    )(feat, feat, deg_cols, n_feats, Wt, b2)


def kernel(n_feats, edge_index, e_weights, W, b):
    src = edge_index[0].astype(jnp.int32)
    dst = edge_index[1].astype(jnp.int32)
    w = e_weights.reshape(E).astype(jnp.float32)
    npd = E2 - E
    srcp = jnp.concatenate([src, jnp.zeros((npd,), jnp.int32)])
    dstp = jnp.concatenate([dst, jnp.full((npd,), NPAD - 1, jnp.int32)])
    wp = jnp.concatenate([w, jnp.zeros((npd,), jnp.float32)])
    pk = jnp.concatenate(
        [srcp.reshape(E2 // C, C),
         dstp.reshape(E2 // C, C),
         lax.bitcast_convert_type(wp, jnp.int32).reshape(E2 // C, C)],
        axis=1)
    pk = jnp.pad(pk, ((0, 4), (0, 0)))
    nf2 = n_feats.reshape(N, 2, HALF).transpose(1, 0, 2).reshape(2 * N, HALF)
    feat = _sc_segment_sum(nf2, pk)
    deg = _sc_degree(dstp)
    deg_cols = deg.reshape(NPAD, 16)
    return _tc_linear(feat, deg_cols, n_feats, W.T, b.reshape(1, D))


# speedup vs baseline: 3.7684x; 1.0013x over previous
import dataclasses
import functools

import jax
import jax.numpy as jnp
from jax import lax
from jax.experimental import pallas as pl
from jax.experimental.pallas import tpu as pltpu
from jax.experimental.pallas import tpu_sc as plsc

N = 10000
E = 160000
D = 256
HALF = 128
NPAD = 10240
C = 80
NSUB = 16
E2 = 163840
EPS = E2 // NSUB
NC = EPS // C
PKROW = 3 * C
RPS = NPAD // NSUB
NHALF = NPAD // 2
DROWS = NHALF * 16 // 128
DPS = DROWS // NSUB
CD = 128
NCD = EPS // CD


def _compiler_params():
    cp = pltpu.CompilerParams()
    if "needs_layout_passes" in pltpu.CompilerParams.__dataclass_fields__:
        cp = dataclasses.replace(cp, needs_layout_passes=False)
    return cp


def _sc_segment_sum(nf2, pk):
    mesh = plsc.VectorSubcoreMesh(core_axis_name="c", subcore_axis_name="s")

    @functools.partial(
        pl.kernel,
        compiler_params=_compiler_params(),
        out_type=jax.ShapeDtypeStruct((2, NPAD, HALF), jnp.float32),
        mesh=mesh,
        scratch_types=[
            pltpu.VMEM((4, PKROW), jnp.int32),
            pltpu.VMEM((4, C), jnp.int32),
            pltpu.VMEM((4, C), jnp.int32),
            pltpu.VMEM((2, C, HALF), jnp.float32),
            pltpu.VMEM((2, C, HALF), jnp.float32),
            pltpu.VMEM_SHARED((NPAD, HALF), jnp.float32),
            pltpu.SemaphoreType.DMA,
            pltpu.SemaphoreType.DMA,
            pltpu.SemaphoreType.DMA,
            pltpu.SemaphoreType.DMA,
            pltpu.SemaphoreType.DMA,
            pltpu.SemaphoreType.DMA,
            pltpu.SemaphoreType.DMA,
            pltpu.SemaphoreType.DMA,
        ],
    )
    def k(nf2_hbm, pk_hbm, feat_hbm,
          idx_v, adj_v, rel_v, rows_v, msg_v, acc_sh,
          si0, si1, si2, si3, sg0, sg1, ss0, ss1):
        cid = lax.axis_index("c")
        sid = lax.axis_index("s")
        zero16 = jnp.zeros((16,), jnp.float32)
        off = cid * N
        row0 = sid * NC
        sem_i = [si0, si1, si2, si3]
        sem_g = [sg0, sg1]
        sem_s = [ss0, ss1]

        def load_idx(kk, b):
            pltpu.async_copy(pk_hbm.at[row0 + kk], idx_v.at[b], sem_i[b])

        def wait_idx(kk, b):
            pltpu.make_async_copy(
                pk_hbm.at[row0 + kk], idx_v.at[b], sem_i[b]).wait()

        def prep(b, gb):
            @pl.loop(0, C, step=16)
            def _(i):
                adj_v[b, pl.ds(i, 16)] = idx_v[b, pl.ds(i, 16)] + off
                rel_v[b, pl.ds(i, 16)] = idx_v[b, pl.ds(C + i, 16)]

            pltpu.async_copy(nf2_hbm.at[adj_v.at[b]], rows_v.at[gb],
                             sem_g[gb])

        def wait_gather(b, gb):
            pltpu.make_async_copy(
                nf2_hbm.at[adj_v.at[b]], rows_v.at[gb], sem_g[gb]).wait()

        def drain_scatter(b, gb):
            pltpu.make_async_copy(
                msg_v.at[gb], acc_sh.at[rel_v.at[b]], sem_s[gb]).wait()

        def finish(b, gb):
            @pl.loop(0, C, step=16)
            def _(i):
                w16 = lax.bitcast_convert_type(
                    idx_v[b, pl.ds(2 * C + i, 16)], jnp.float32)
                for r in range(16):
                    wv = jnp.full((16,), w16[r], jnp.float32)
                    for j in range(HALF // 16):
                        msg_v[gb, i + r, pl.ds(j * 16, 16)] = (
                            rows_v[gb, i + r, pl.ds(j * 16, 16)] * wv)

            pltpu.async_copy(msg_v.at[gb], acc_sh.at[rel_v.at[b]],
                             sem_s[gb], add=True)

        @pl.loop(0, C)
        def _(r):
            for g in range(2):
                for j in range(HALF // 16):
                    msg_v[g, r, pl.ds(j * 16, 16)] = zero16

        zi16 = jnp.zeros((16,), jnp.int32)

        @pl.loop(0, C, step=16)
        def _(i):
            rel_v[2, pl.ds(i, 16)] = zi16
            rel_v[3, pl.ds(i, 16)] = zi16

        abase = sid * RPS
        for t in range(RPS // C):
            pltpu.sync_copy(msg_v.at[0], acc_sh.at[pl.ds(abase + t * C, C)])

        plsc.subcore_barrier()

        pltpu.async_copy(msg_v.at[0], acc_sh.at[rel_v.at[2]], sem_s[0],
                         add=True)
        pltpu.async_copy(msg_v.at[1], acc_sh.at[rel_v.at[3]], sem_s[1],
                         add=True)

        load_idx(0, 0)
        wait_idx(0, 0)
        prep(0, 0)
        load_idx(1, 1)

        @pl.loop(0, NC // 4)
        def _(mm):
            kk = mm * 4
            for j in range(4):
                b, nb = j % 4, (j + 1) % 4
                gb, ngb = j % 2, (j + 1) % 2
                wait_idx(kk + j + 1, nb)
                prep(nb, ngb)
                load_idx(kk + j + 2, (j + 2) % 4)
                wait_gather(b, gb)
                drain_scatter((j + 2) % 4, gb)
                finish(b, gb)

        drain_scatter(2, 0)
        drain_scatter(3, 1)
        wait_gather(0, 0)
        wait_idx(NC + 1, 1)

        plsc.subcore_barrier()

        for t in range(RPS // C):
            pltpu.sync_copy(
                acc_sh.at[pl.ds(abase + t * C, C)],
                feat_hbm.at[cid].at[pl.ds(abase + t * C, C)])

    return k(nf2, pk)


def _sc_degree(dstp):
    mesh = plsc.VectorSubcoreMesh(core_axis_name="c", subcore_axis_name="s")

    @functools.partial(
        pl.kernel,
        compiler_params=_compiler_params(),
        out_type=jax.ShapeDtypeStruct((2, DROWS, 128), jnp.float32),
        mesh=mesh,
        scratch_types=[
            pltpu.VMEM((EPS,), jnp.int32),
            pltpu.VMEM((DROWS, 128), jnp.float32),
            pltpu.VMEM((DROWS // 128, 128), jnp.int32),
            pltpu.VMEM_SHARED((DROWS, 128), jnp.float32),
        ],
    )
    def k(dst_hbm, deg_hbm, dst_v, degh_v, id_v, deg_sh):
        cid = lax.axis_index("c")
        sid = lax.axis_index("s")
        zero16 = jnp.zeros((16,), jnp.float32)
        one16 = jnp.ones((16,), jnp.float32)
        lane16 = jnp.arange(16, dtype=jnp.int32)
        ebase = sid * EPS
        deg_lo = cid * NHALF

        @pl.loop(0, DROWS)
        def _(r):
            for j in range(128 // 16):
                degh_v[r, pl.ds(j * 16, 16)] = zero16

        for j in range(DROWS // 128):
            for t in range(128 // 16):
                id_v[j, pl.ds(t * 16, 16)] = lane16 + (j * 128 + t * 16)

        pltpu.sync_copy(degh_v.at[pl.ds(0, DPS)],
                        deg_sh.at[pl.ds(sid * DPS, DPS)])

        plsc.subcore_barrier()

        pltpu.sync_copy(dst_hbm.at[pl.ds(ebase, EPS)], dst_v)

        @pl.loop(0, EPS, step=16)
        def _(i):
            d16 = dst_v[pl.ds(i, 16)]
            rel = d16 - deg_lo
            mask = (rel >= 0) & (rel < NHALF)
            relc = jnp.where(mask, rel, 0)
            flat = relc * 16 + lane16
            plsc.addupdate_scatter(
                degh_v, [flat >> 7, flat & 127], one16, mask=mask)

        for j in range(DROWS // 128):
            pltpu.sync_copy(degh_v.at[pl.ds(j * 128, 128)],
                            deg_sh.at[id_v.at[j]], add=True)

        plsc.subcore_barrier()

        pltpu.sync_copy(deg_sh.at[pl.ds(sid * DPS, DPS)],
                        deg_hbm.at[cid].at[pl.ds(sid * DPS, DPS)])

    return k(dstp)


_BLK = 1000


def _tc_linear(feat, deg_cols, n_feats, Wt, b2):
    def body(acc0_ref, acc1_ref, deg_ref, nf_ref, wt_ref, b_ref, out_ref):
        deg = jnp.sum(deg_ref[...], axis=1, keepdims=True)
        inv = 1.0 / jnp.maximum(deg, 1.0)
        ha = acc0_ref[...] * inv
        hb = acc1_ref[...] * inv
        wt = wt_ref[...]
        o = jnp.dot(ha, wt[:HALF], preferred_element_type=jnp.float32)
        o = o + jnp.dot(hb, wt[HALF:D], preferred_element_type=jnp.float32)
        o = o + jnp.dot(nf_ref[...], wt[D:], preferred_element_type=jnp.float32)
        out_ref[...] = jnp.maximum(o + b_ref[...], 0.0)

    return pl.pallas_call(
        body,
        grid=(N // _BLK,),
        in_specs=[
            pl.BlockSpec((None, _BLK, HALF), lambda i: (0, i, 0)),
            pl.BlockSpec((None, _BLK, HALF), lambda i: (1, i, 0)),
            pl.BlockSpec((_BLK, 16), lambda i: (i, 0)),
            pl.BlockSpec((_BLK, D), lambda i: (i, 0)),
            pl.BlockSpec((2 * D, D), lambda i: (0, 0)),
            pl.BlockSpec((1, D), lambda i: (0, 0)),
        ],
        out_specs=pl.BlockSpec((_BLK, D), lambda i: (i, 0)),
        out_shape=jax.ShapeDtypeStruct((N, D), jnp.float32),
    )(feat, feat, deg_cols, n_feats, Wt, b2)


def kernel(n_feats, edge_index, e_weights, W, b):
    src = edge_index[0].astype(jnp.int32)
    dst = edge_index[1].astype(jnp.int32)
    w = e_weights.reshape(E).astype(jnp.float32)
    npd = E2 - E
    srcp = jnp.concatenate([src, jnp.zeros((npd,), jnp.int32)])
    dstp = jnp.concatenate([dst, jnp.full((npd,), NPAD - 1, jnp.int32)])
    wp = jnp.concatenate([w, jnp.zeros((npd,), jnp.float32)])
    pk = jnp.concatenate(
        [srcp.reshape(E2 // C, C),
         dstp.reshape(E2 // C, C),
         lax.bitcast_convert_type(wp, jnp.int32).reshape(E2 // C, C)],
        axis=1)
    pk = jnp.pad(pk, ((0, 4), (0, 0)))
    nf2 = n_feats.reshape(N, 2, HALF).transpose(1, 0, 2).reshape(2 * N, HALF)
    feat = _sc_segment_sum(nf2, pk)
    deg = _sc_degree(dstp)
    deg_cols = deg.reshape(NPAD, 16)
    return _tc_linear(feat, deg_cols, n_feats, W.T, b.reshape(1, D))
